# baseline (speedup 1.0000x reference)
import functools

import jax
import jax.numpy as jnp
from jax import lax
from jax.experimental import pallas as pl
from jax.experimental.pallas import tpu as pltpu
from jax.experimental.pallas import tpu_sc as plsc

N_NODES = 10000
N_EDGES = 320000
D = 128
DH = D // 2

NC = 2
NS = 16
NW = NC * NS

K = 64
NBK = 10
CH = K * NBK
NCH = 16
EPT = CH * NCH
E_PAD = NW * EPT
PAD_COL = N_NODES + 200

N_PAD = 10240
DEG_SL = N_PAD // NS


def _rsqrt16(d):
    i = lax.bitcast_convert_type(d, jnp.int32)
    magic = jnp.full((16,), 0x5F3759DF, jnp.int32)
    y = lax.bitcast_convert_type(magic - lax.shift_right_logical(i, 1), jnp.float32)
    for _ in range(3):
        y = y * (1.5 - 0.5 * d * y * y)
    return jnp.where(d > 0.5, y, 0.0)


def _sc_body(h0_hbm, h1_hbm, row_hbm, col_hbm, cns_hbm, out_hbm,
             rowb0, colb0, cnsb0, normb0, rowb1, colb1, cnsb1, normb1,
             disv, zv, ov, degb, gbufa, gbufb, sbufa, sbufb,
             deg_sh, out_sh, esem, gsem0, gsem1, ssem0, ssem1):
    c = lax.axis_index("c")
    s = lax.axis_index("s")
    wid = c * NS + s
    gbufs = (gbufa, gbufb)
    sbufs = (sbufa, sbufb)
    gsems = (gsem0, gsem1)
    ssems = (ssem0, ssem1)

    one16 = jnp.ones((16,), jnp.float32)
    for j in range(K // 16):
        ov[pl.ds(j * 16, 16)] = one16

    def _zv_zero(i, _):
        zv[pl.ds(i * 16, 16)] = jnp.zeros((16,), jnp.float32)
        return _
    lax.fori_loop(0, DEG_SL // 16, _zv_zero, None)

    def _gbufa_zero(e, _):
        for j in range(DH // 16):
            gbufa[e, pl.ds(j * 16, 16)] = jnp.zeros((16,), jnp.float32)
        return _
    lax.fori_loop(0, K, _gbufa_zero, None)

    obase = s * DEG_SL

    def _zero_out_sh():
        def _fire(q, _):
            pltpu.async_copy(gbufa, out_sh.at[pl.ds(obase + q * K, K)], esem)
            return _
        lax.fori_loop(0, DEG_SL // K, _fire, None)

        def _drain(q, _):
            pltpu.make_async_copy(gbufa, out_sh.at[pl.ds(obase, K)], esem).wait()
            return _
        lax.fori_loop(0, DEG_SL // K, _drain, None)

    with jax.named_scope("zero"):
        pltpu.sync_copy(zv, deg_sh.at[pl.ds(s * DEG_SL, DEG_SL)])
        _zero_out_sh()
        plsc.subcore_barrier()

    with jax.named_scope("deg"):
      for j in range(2):
        pltpu.sync_copy(col_hbm.at[2 * s + j], degb)

        def _deg_fire(i, _):
            ch = i // NBK
            b = i - ch * NBK
            pltpu.async_copy(ov, deg_sh.at[degb.at[ch, b]], esem, add=True)
            return _
        lax.fori_loop(0, NCH * NBK, _deg_fire, None)

        def _deg_drain(i, _):
            pltpu.make_async_copy(ov, deg_sh.at[degb.at[0, 0]], esem).wait()
            return _
        lax.fori_loop(0, NCH * NBK, _deg_drain, None)
    plsc.subcore_barrier()

    with jax.named_scope("dis"):
        doff = s * DEG_SL
        pltpu.sync_copy(deg_sh.at[pl.ds(doff, DEG_SL)], zv)

        def _dis(i, _):
            sl = pl.ds(i * 16, 16)
            zv[sl] = _rsqrt16(zv[sl])
            return _
        lax.fori_loop(0, DEG_SL // 16, _dis, None)
        pltpu.sync_copy(zv, deg_sh.at[pl.ds(doff, DEG_SL)])
        plsc.subcore_barrier()

        pltpu.sync_copy(deg_sh, disv)

    def _norm_into(rowb, colb, cnsb, normb):
        def _norm(i, _):
            for g in range(K // 16):
                sl = pl.ds(g * 16, 16)
                dr = plsc.load_gather(disv, [rowb[i, sl]])
                dc = plsc.load_gather(disv, [colb[i, sl]])
                normb[i, sl] = dr * dc * jnp.exp(cnsb[i, sl])
            return _
        lax.fori_loop(0, NBK, _norm, None)

    bufsets = ((rowb0, colb0, cnsb0, normb0), (rowb1, colb1, cnsb1, normb1))

    def _half_pass(href):
        def _scale(p, normb, b):
            gbuf, sbuf = gbufs[p], sbufs[p]
            for g in range(K // 16):
                nv = normb[b, pl.ds(g * 16, 16)]
                for e in range(16):
                    r = g * 16 + e
                    sc = nv[e]
                    for jj in range(DH // 16):
                        sl = pl.ds(jj * 16, 16)
                        sbuf[r, sl] = gbuf[r, sl] * sc

        def _gather(p, rowb, b):
            pltpu.async_copy(href.at[rowb.at[b]], gbufs[p], gsems[p])

        def _pair(rowb, colb, normb, bb, next_gather):
            for p in range(2):
                b = bb + p
                pltpu.make_async_copy(href.at[rowb.at[0]], gbufs[p], gsems[p]).wait()
                pltpu.make_async_copy(sbufs[p], out_sh.at[colb.at[0]], ssems[p]).wait()
                _scale(p, normb, b)
                pltpu.async_copy(sbufs[p], out_sh.at[colb.at[b]], ssems[p], add=True)
                next_gather(p)

        def _chunk_code(par, ch, has_next):
            rowb, colb, cnsb, normb = bufsets[par]
            rowbn, colbn, cnsbn, normbn = bufsets[1 - par]

            _pair(rowb, colb, normb, 0, lambda p: _gather(p, rowb, 2 + p))

            @pl.when(has_next)
            def _():
                pltpu.async_copy(row_hbm.at[wid, ch + 1], rowbn, esem)
                pltpu.async_copy(col_hbm.at[wid, ch + 1], colbn, esem)
                pltpu.async_copy(cns_hbm.at[wid, ch + 1], cnsbn, esem)

            def _c(q, _):
                _pair(rowb, colb, normb, 2 * q,
                      lambda p: _gather(p, rowb, 2 * q + 2 + p))
                return _
            lax.fori_loop(1, NBK // 2 - 1, _c, None)

            @pl.when(has_next)
            def _():
                pltpu.make_async_copy(row_hbm.at[wid, 0], rowbn, esem).wait()
                pltpu.make_async_copy(col_hbm.at[wid, 0], colbn, esem).wait()
                pltpu.make_async_copy(cns_hbm.at[wid, 0], cnsbn, esem).wait()
                _norm_into(rowbn, colbn, cnsbn, normbn)

            def _next_g(p):
                @pl.when(has_next)
                def _():
                    _gather(p, rowbn, p)
            _pair(rowb, colb, normb, NBK - 2, _next_g)

        def _sbuf_zero(e, _):
            for j in range(DH // 16):
                sbufa[e, pl.ds(j * 16, 16)] = jnp.zeros((16,), jnp.float32)
                sbufb[e, pl.ds(j * 16, 16)] = jnp.zeros((16,), jnp.float32)
            return _
        lax.fori_loop(0, K, _sbuf_zero, None)
        pltpu.sync_copy(row_hbm.at[wid, 0], rowb0)
        pltpu.sync_copy(col_hbm.at[wid, 0], colb0)
        pltpu.sync_copy(cns_hbm.at[wid, 0], cnsb0)
        pltpu.async_copy(sbufa, out_sh.at[colb0.at[0]], ssem0, add=True)
        pltpu.async_copy(sbufb, out_sh.at[colb0.at[0]], ssem1, add=True)
        _norm_into(rowb0, colb0, cnsb0, normb0)
        _gather(0, rowb0, 0)
        _gather(1, rowb0, 1)

        def _chunk_pair(i, _):
            _chunk_code(0, 2 * i, True)
            _chunk_code(1, 2 * i + 1, i < NCH // 2 - 1)
            return _
        lax.fori_loop(0, NCH // 2, _chunk_pair, None)

        pltpu.make_async_copy(sbufa, out_sh.at[colb1.at[0]], ssem0).wait()
        pltpu.make_async_copy(sbufb, out_sh.at[colb1.at[0]], ssem1).wait()

    for hf, href in enumerate((h0_hbm, h1_hbm)):
        with jax.named_scope(f"half{hf}"):
            _half_pass(href)
            plsc.subcore_barrier()
        with jax.named_scope(f"dump{hf}"):
            pltpu.sync_copy(out_sh.at[pl.ds(obase, DEG_SL)],
                            out_hbm.at[pl.ds((c * 2 + hf) * N_PAD + obase, DEG_SL)])
        if hf == 0:
            def _gbufa_rezero(e, _):
                for j in range(DH // 16):
                    gbufa[e, pl.ds(j * 16, 16)] = jnp.zeros((16,), jnp.float32)
                return _
            lax.fori_loop(0, K, _gbufa_rezero, None)
            _zero_out_sh()
            plsc.subcore_barrier()


_sc_propagate = functools.partial(
    pl.kernel,
    out_type=jax.ShapeDtypeStruct((NC * 2 * N_PAD, DH), jnp.float32),
    mesh=plsc.VectorSubcoreMesh(core_axis_name="c", subcore_axis_name="s"),
    compiler_params=pltpu.CompilerParams(needs_layout_passes=False,
                                         use_tc_tiling_on_sc=False),
    scratch_types=[
        pltpu.VMEM((NBK, K), jnp.int32),
        pltpu.VMEM((NBK, K), jnp.int32),
        pltpu.VMEM((NBK, K), jnp.float32),
        pltpu.VMEM((NBK, K), jnp.float32),
        pltpu.VMEM((NBK, K), jnp.int32),
        pltpu.VMEM((NBK, K), jnp.int32),
        pltpu.VMEM((NBK, K), jnp.float32),
        pltpu.VMEM((NBK, K), jnp.float32),
        pltpu.VMEM((N_PAD,), jnp.float32),
        pltpu.VMEM((DEG_SL,), jnp.float32),
        pltpu.VMEM((K,), jnp.float32),
        pltpu.VMEM((NCH, NBK, K), jnp.int32),
        pltpu.VMEM((K, DH), jnp.float32),
        pltpu.VMEM((K, DH), jnp.float32),
        pltpu.VMEM((K, DH), jnp.float32),
        pltpu.VMEM((K, DH), jnp.float32),
        pltpu.VMEM_SHARED((N_PAD,), jnp.float32),
        pltpu.VMEM_SHARED((N_PAD, DH), jnp.float32),
        pltpu.SemaphoreType.DMA,
        pltpu.SemaphoreType.DMA,
        pltpu.SemaphoreType.DMA,
        pltpu.SemaphoreType.DMA,
        pltpu.SemaphoreType.DMA,
    ],
)(_sc_body)


def _mm_body(x_ref, w_ref, o_ref):
    o_ref[...] = lax.dot_general(
        x_ref[...], w_ref[...], (((1,), (1,)), ((), ())),
        preferred_element_type=jnp.float32)


def _comb_body(p00, p01, p10, p11, b_ref, o_ref):
    o_ref[:, :DH] = p00[0, 0] + p10[0, 0] + b_ref[0, :DH]
    o_ref[:, DH:] = p01[0, 0] + p11[0, 0] + b_ref[0, DH:]


def kernel(x, edge_index, cns, W, bias):
    n, d_in = x.shape
    d_out = W.shape[0]
    nblk = 10
    h = pl.pallas_call(
        _mm_body,
        grid=(nblk,),
        in_specs=[
            pl.BlockSpec((n // nblk, d_in), lambda i: (i, 0)),
            pl.BlockSpec((d_out, d_in), lambda i: (0, 0)),
        ],
        out_specs=pl.BlockSpec((n // nblk, d_out), lambda i: (i, 0)),
        out_shape=jax.ShapeDtypeStruct((n, d_out), jnp.float32),
    )(x, W)

    n_edges = edge_index.shape[1]
    pad = E_PAD - n_edges
    row4 = jnp.concatenate(
        [edge_index[0], jnp.zeros((pad,), jnp.int32)]).reshape(NW, NCH, NBK, K)
    col4 = jnp.concatenate(
        [edge_index[1], jnp.full((pad,), PAD_COL, jnp.int32)]).reshape(NW, NCH, NBK, K)
    cns4 = jnp.concatenate(
        [cns, jnp.full((pad,), -1e4, cns.dtype)]).reshape(NW, NCH, NBK, K)
    h0 = h[:, :DH]
    h1 = h[:, DH:]
    part = _sc_propagate(h0, h1, row4, col4, cns4).reshape(NC, 2, N_PAD, DH)

    out = pl.pallas_call(
        _comb_body,
        grid=(nblk,),
        in_specs=[
            pl.BlockSpec((1, 1, n // nblk, DH), lambda i: (0, 0, i, 0)),
            pl.BlockSpec((1, 1, n // nblk, DH), lambda i: (0, 1, i, 0)),
            pl.BlockSpec((1, 1, n // nblk, DH), lambda i: (1, 0, i, 0)),
            pl.BlockSpec((1, 1, n // nblk, DH), lambda i: (1, 1, i, 0)),
            pl.BlockSpec((1, d_out), lambda i: (0, 0)),
        ],
        out_specs=pl.BlockSpec((n // nblk, d_out), lambda i: (i, 0)),
        out_shape=jax.ShapeDtypeStruct((n, d_out), jnp.float32),
    )(part, part, part, part, bias.reshape(1, d_out))
    return out


# speedup vs baseline: 1.0823x; 1.0823x over previous
import functools

import jax
import jax.numpy as jnp
from jax import lax
from jax.experimental import pallas as pl
from jax.experimental.pallas import tpu as pltpu
from jax.experimental.pallas import tpu_sc as plsc

N_NODES = 10000
N_EDGES = 320000
D = 128
DH = D // 2

NC = 2
NS = 16
NW = NC * NS

K = 64
NBK = 10
CH = K * NBK
NCHT = 512
CPT0 = 24
CPT1 = 8
E_PAD = NCHT * CH
PAD_COL = N_NODES + 200

N_PAD = 10240
DEG_SL = N_PAD // NS


def _rsqrt16(d):
    i = lax.bitcast_convert_type(d, jnp.int32)
    magic = jnp.full((16,), 0x5F3759DF, jnp.int32)
    y = lax.bitcast_convert_type(magic - lax.shift_right_logical(i, 1), jnp.float32)
    for _ in range(3):
        y = y * (1.5 - 0.5 * d * y * y)
    return jnp.where(d > 0.5, y, 0.0)


def _sc_body(h0_hbm, h1_hbm, row_hbm, col_hbm, cns_hbm, out_hbm,
             rowb0, colb0, cnsb0, normb0, rowb1, colb1, cnsb1, normb1,
             disv, zv, ov, degb, gbufa, gbufb, sbufa, sbufb,
             deg_sh, out_sh, esem, gsem0, gsem1, ssem0, ssem1):
    c = lax.axis_index("c")
    s = lax.axis_index("s")
    base = jnp.where(c == 0, CPT0 * s, NS * CPT0 + CPT1 * s)
    cnt_pairs = jnp.where(c == 0, CPT0 // 2, CPT1 // 2)
    gbufs = (gbufa, gbufb)
    sbufs = (sbufa, sbufb)
    gsems = (gsem0, gsem1)
    ssems = (ssem0, ssem1)

    one16 = jnp.ones((16,), jnp.float32)
    for j in range(K // 16):
        ov[pl.ds(j * 16, 16)] = one16

    def _zv_zero(i, _):
        zv[pl.ds(i * 16, 16)] = jnp.zeros((16,), jnp.float32)
        return _
    lax.fori_loop(0, DEG_SL // 16, _zv_zero, None)

    def _gbufa_zero(e, _):
        for j in range(DH // 16):
            gbufa[e, pl.ds(j * 16, 16)] = jnp.zeros((16,), jnp.float32)
        return _
    lax.fori_loop(0, K, _gbufa_zero, None)

    obase = s * DEG_SL

    def _zero_out_sh():
        def _fire(q, _):
            pltpu.async_copy(gbufa, out_sh.at[pl.ds(obase + q * K, K)], esem)
            return _
        lax.fori_loop(0, DEG_SL // K, _fire, None)

        def _drain(q, _):
            pltpu.make_async_copy(gbufa, out_sh.at[pl.ds(obase, K)], esem).wait()
            return _
        lax.fori_loop(0, DEG_SL // K, _drain, None)

    with jax.named_scope("zero"):
        pltpu.sync_copy(zv, deg_sh.at[pl.ds(s * DEG_SL, DEG_SL)])
        _zero_out_sh()
        plsc.subcore_barrier()

    with jax.named_scope("deg"):
        pltpu.sync_copy(col_hbm.at[pl.ds(32 * s, 32)], degb)

        def _deg_fire(i, _):
            ch = i // NBK
            b = i - ch * NBK
            pltpu.async_copy(ov, deg_sh.at[degb.at[ch, b]], esem, add=True)
            return _
        lax.fori_loop(0, 32 * NBK, _deg_fire, None)

        def _deg_drain(i, _):
            pltpu.make_async_copy(ov, deg_sh.at[degb.at[0, 0]], esem).wait()
            return _
        lax.fori_loop(0, 32 * NBK, _deg_drain, None)
    plsc.subcore_barrier()

    with jax.named_scope("dis"):
        doff = s * DEG_SL
        pltpu.sync_copy(deg_sh.at[pl.ds(doff, DEG_SL)], zv)

        def _dis(i, _):
            sl = pl.ds(i * 16, 16)
            zv[sl] = _rsqrt16(zv[sl])
            return _
        lax.fori_loop(0, DEG_SL // 16, _dis, None)
        pltpu.sync_copy(zv, deg_sh.at[pl.ds(doff, DEG_SL)])
        plsc.subcore_barrier()

        pltpu.sync_copy(deg_sh, disv)

    def _norm_into(rowb, colb, cnsb, normb):
        def _norm(i, _):
            for g in range(K // 16):
                sl = pl.ds(g * 16, 16)
                dr = plsc.load_gather(disv, [rowb[i, sl]])
                dc = plsc.load_gather(disv, [colb[i, sl]])
                normb[i, sl] = dr * dc * jnp.exp(cnsb[i, sl])
            return _
        lax.fori_loop(0, NBK, _norm, None)

    bufsets = ((rowb0, colb0, cnsb0, normb0), (rowb1, colb1, cnsb1, normb1))

    def _half_pass(href):
        def _scale(p, normb, b):
            gbuf, sbuf = gbufs[p], sbufs[p]
            for g in range(K // 16):
                nv = normb[b, pl.ds(g * 16, 16)]
                for e in range(16):
                    r = g * 16 + e
                    sc = nv[e]
                    for jj in range(DH // 16):
                        sl = pl.ds(jj * 16, 16)
                        sbuf[r, sl] = gbuf[r, sl] * sc

        def _gather(p, rowb, b):
            pltpu.async_copy(href.at[rowb.at[b]], gbufs[p], gsems[p])

        def _pair(rowb, colb, normb, bb, next_gather):
            for p in range(2):
                b = bb + p
                pltpu.make_async_copy(href.at[rowb.at[0]], gbufs[p], gsems[p]).wait()
                pltpu.make_async_copy(sbufs[p], out_sh.at[colb.at[0]], ssems[p]).wait()
                _scale(p, normb, b)
                pltpu.async_copy(sbufs[p], out_sh.at[colb.at[b]], ssems[p], add=True)
                next_gather(p)

        def _chunk_code(par, ch, has_next):
            rowb, colb, cnsb, normb = bufsets[par]
            rowbn, colbn, cnsbn, normbn = bufsets[1 - par]

            _pair(rowb, colb, normb, 0, lambda p: _gather(p, rowb, 2 + p))

            @pl.when(has_next)
            def _():
                pltpu.async_copy(row_hbm.at[ch + 1], rowbn, esem)
                pltpu.async_copy(col_hbm.at[ch + 1], colbn, esem)
                pltpu.async_copy(cns_hbm.at[ch + 1], cnsbn, esem)

            def _c(q, _):
                _pair(rowb, colb, normb, 2 * q,
                      lambda p: _gather(p, rowb, 2 * q + 2 + p))
                return _
            lax.fori_loop(1, NBK // 2 - 1, _c, None)

            @pl.when(has_next)
            def _():
                pltpu.make_async_copy(row_hbm.at[base], rowbn, esem).wait()
                pltpu.make_async_copy(col_hbm.at[base], colbn, esem).wait()
                pltpu.make_async_copy(cns_hbm.at[base], cnsbn, esem).wait()
                _norm_into(rowbn, colbn, cnsbn, normbn)

            def _next_g(p):
                @pl.when(has_next)
                def _():
                    _gather(p, rowbn, p)
            _pair(rowb, colb, normb, NBK - 2, _next_g)

        def _sbuf_zero(e, _):
            for j in range(DH // 16):
                sbufa[e, pl.ds(j * 16, 16)] = jnp.zeros((16,), jnp.float32)
                sbufb[e, pl.ds(j * 16, 16)] = jnp.zeros((16,), jnp.float32)
            return _
        lax.fori_loop(0, K, _sbuf_zero, None)
        pltpu.sync_copy(row_hbm.at[base], rowb0)
        pltpu.sync_copy(col_hbm.at[base], colb0)
        pltpu.sync_copy(cns_hbm.at[base], cnsb0)
        pltpu.async_copy(sbufa, out_sh.at[colb0.at[0]], ssem0, add=True)
        pltpu.async_copy(sbufb, out_sh.at[colb0.at[0]], ssem1, add=True)
        _norm_into(rowb0, colb0, cnsb0, normb0)
        _gather(0, rowb0, 0)
        _gather(1, rowb0, 1)

        def _chunk_pair(i, _):
            _chunk_code(0, base + 2 * i, True)
            _chunk_code(1, base + 2 * i + 1, i < cnt_pairs - 1)
            return _
        lax.fori_loop(0, cnt_pairs, _chunk_pair, None)

        pltpu.make_async_copy(sbufa, out_sh.at[colb1.at[0]], ssem0).wait()
        pltpu.make_async_copy(sbufb, out_sh.at[colb1.at[0]], ssem1).wait()

    for hf, href in enumerate((h0_hbm, h1_hbm)):
        with jax.named_scope(f"half{hf}"):
            _half_pass(href)
            plsc.subcore_barrier()
        with jax.named_scope(f"dump{hf}"):
            pltpu.sync_copy(out_sh.at[pl.ds(obase, DEG_SL)],
                            out_hbm.at[pl.ds((c * 2 + hf) * N_PAD + obase, DEG_SL)])
        if hf == 0:
            def _gbufa_rezero(e, _):
                for j in range(DH // 16):
                    gbufa[e, pl.ds(j * 16, 16)] = jnp.zeros((16,), jnp.float32)
                return _
            lax.fori_loop(0, K, _gbufa_rezero, None)
            _zero_out_sh()
            plsc.subcore_barrier()


_sc_propagate = functools.partial(
    pl.kernel,
    out_type=jax.ShapeDtypeStruct((NC * 2 * N_PAD, DH), jnp.float32),
    mesh=plsc.VectorSubcoreMesh(core_axis_name="c", subcore_axis_name="s"),
    compiler_params=pltpu.CompilerParams(needs_layout_passes=False,
                                         use_tc_tiling_on_sc=False),
    scratch_types=[
        pltpu.VMEM((NBK, K), jnp.int32),
        pltpu.VMEM((NBK, K), jnp.int32),
        pltpu.VMEM((NBK, K), jnp.float32),
        pltpu.VMEM((NBK, K), jnp.float32),
        pltpu.VMEM((NBK, K), jnp.int32),
        pltpu.VMEM((NBK, K), jnp.int32),
        pltpu.VMEM((NBK, K), jnp.float32),
        pltpu.VMEM((NBK, K), jnp.float32),
        pltpu.VMEM((N_PAD,), jnp.float32),
        pltpu.VMEM((DEG_SL,), jnp.float32),
        pltpu.VMEM((K,), jnp.float32),
        pltpu.VMEM((32, NBK, K), jnp.int32),
        pltpu.VMEM((K, DH), jnp.float32),
        pltpu.VMEM((K, DH), jnp.float32),
        pltpu.VMEM((K, DH), jnp.float32),
        pltpu.VMEM((K, DH), jnp.float32),
        pltpu.VMEM_SHARED((N_PAD,), jnp.float32),
        pltpu.VMEM_SHARED((N_PAD, DH), jnp.float32),
        pltpu.SemaphoreType.DMA,
        pltpu.SemaphoreType.DMA,
        pltpu.SemaphoreType.DMA,
        pltpu.SemaphoreType.DMA,
        pltpu.SemaphoreType.DMA,
    ],
)(_sc_body)


def _mm_body(x_ref, w_ref, o_ref):
    o_ref[...] = lax.dot_general(
        x_ref[...], w_ref[...], (((1,), (1,)), ((), ())),
        preferred_element_type=jnp.float32)


def _comb_body(p00, p01, p10, p11, b_ref, o_ref):
    o_ref[:, :DH] = p00[0, 0] + p10[0, 0] + b_ref[0, :DH]
    o_ref[:, DH:] = p01[0, 0] + p11[0, 0] + b_ref[0, DH:]


def kernel(x, edge_index, cns, W, bias):
    n, d_in = x.shape
    d_out = W.shape[0]
    nblk = 10
    h = pl.pallas_call(
        _mm_body,
        grid=(nblk,),
        in_specs=[
            pl.BlockSpec((n // nblk, d_in), lambda i: (i, 0)),
            pl.BlockSpec((d_out, d_in), lambda i: (0, 0)),
        ],
        out_specs=pl.BlockSpec((n // nblk, d_out), lambda i: (i, 0)),
        out_shape=jax.ShapeDtypeStruct((n, d_out), jnp.float32),
    )(x, W)

    n_edges = edge_index.shape[1]
    pad = E_PAD - n_edges
    row4 = jnp.concatenate(
        [edge_index[0], jnp.zeros((pad,), jnp.int32)]).reshape(NCHT, NBK, K)
    col4 = jnp.concatenate(
        [edge_index[1], jnp.full((pad,), PAD_COL, jnp.int32)]).reshape(NCHT, NBK, K)
    cns4 = jnp.concatenate(
        [cns, jnp.full((pad,), -1e4, cns.dtype)]).reshape(NCHT, NBK, K)
    h0 = h[:, :DH]
    h1 = h[:, DH:]
    part = _sc_propagate(h0, h1, row4, col4, cns4).reshape(NC, 2, N_PAD, DH)

    out = pl.pallas_call(
        _comb_body,
        grid=(nblk,),
        in_specs=[
            pl.BlockSpec((1, 1, n // nblk, DH), lambda i: (0, 0, i, 0)),
            pl.BlockSpec((1, 1, n // nblk, DH), lambda i: (0, 1, i, 0)),
            pl.BlockSpec((1, 1, n // nblk, DH), lambda i: (1, 0, i, 0)),
            pl.BlockSpec((1, 1, n // nblk, DH), lambda i: (1, 1, i, 0)),
            pl.BlockSpec((1, d_out), lambda i: (0, 0)),
        ],
        out_specs=pl.BlockSpec((n // nblk, d_out), lambda i: (i, 0)),
        out_shape=jax.ShapeDtypeStruct((n, d_out), jnp.float32),
    )(part, part, part, part, bias.reshape(1, d_out))
    return out
